# baseline (speedup 1.0000x reference)
import functools

import jax
import jax.numpy as jnp
from jax import lax
from jax.experimental import pallas as pl
from jax.experimental.pallas import tpu as pltpu
from jax.experimental.pallas import tpu_sc as plsc

N = 10000
E = 320000
DR = 40
D = 48
K = 10
P = 10240
NW = 32
BUCK = P // NW
CHUNK = 128
ECHUNKS = 2560
EPAD = ECHUNKS * CHUNK
SBLK = 80
NCAP_CH = 88
NCAP = NCAP_CH * CHUNK
PAD_NODE = N + 100

_mesh = plsc.VectorSubcoreMesh(core_axis_name="c", subcore_axis_name="s")
_scparams = pltpu.CompilerParams(use_tc_tiling_on_sc=False,
                                 needs_layout_passes=False)

@functools.partial(
    pl.kernel,
    out_type=[jax.ShapeDtypeStruct((NW, NCAP), jnp.int32),
              jax.ShapeDtypeStruct((NW, NCAP), jnp.int32)],
    mesh=_mesh,
    scratch_types=[
        pltpu.VMEM((SBLK, CHUNK), jnp.int32),
        pltpu.VMEM((SBLK, CHUNK), jnp.int32),
        pltpu.VMEM((NCAP + 16,), jnp.int32),
        pltpu.VMEM((NCAP + 16,), jnp.int32),
    ],
    compiler_params=_scparams,
)
def _sc_prep(srcc_hbm, dstc_hbm, bsrc_hbm, bslot_hbm,
             stage_s, stage_d, bsrc_v, bslot_v):
    c = lax.axis_index("c")
    s = lax.axis_index("s")
    w = s * 2 + c
    wbase = w * BUCK

    def fill(i, carry):
        bsrc_v[pl.ds(16 * i, 16)] = jnp.full((16,), PAD_NODE, jnp.int32)
        bslot_v[pl.ds(16 * i, 16)] = jnp.zeros((16,), jnp.int32)
        return carry

    lax.fori_loop(0, NCAP // 16, fill, 0)

    def block_body(t, cnt):
        pltpu.sync_copy(srcc_hbm.at[pl.ds(t * SBLK, SBLK)], stage_s)
        pltpu.sync_copy(dstc_hbm.at[pl.ds(t * SBLK, SBLK)], stage_d)

        def grp(i, cnt):
            r = i >> 3
            q = (i & 7) * 16
            sv = stage_s[r, pl.ds(q, 16)]
            dv = stage_d[r, pl.ds(q, 16)]
            bkt = (dv * 6554) >> 21
            mask = bkt == w
            mi = mask.astype(jnp.int32)
            incl = plsc.cumsum(mi)
            pos = cnt + incl - mi
            dump = NCAP + lax.iota(jnp.int32, 16)
            idx = jnp.where(mask, pos, dump)
            plsc.store_scatter(bsrc_v, [idx], sv)
            plsc.store_scatter(bslot_v, [idx], dv - wbase)
            return cnt + incl[15]

        return lax.fori_loop(0, SBLK * 8, grp, cnt)

    lax.fori_loop(0, ECHUNKS // SBLK, block_body, 0)
    pltpu.sync_copy(bsrc_v.at[pl.ds(0, NCAP)], bsrc_hbm.at[w])
    pltpu.sync_copy(bslot_v.at[pl.ds(0, NCAP)], bslot_hbm.at[w])


@functools.partial(
    pl.kernel,
    out_type=[jax.ShapeDtypeStruct((P, D), jnp.float32),
              jax.ShapeDtypeStruct((P, D), jnp.float32)],
    mesh=_mesh,
    scratch_types=[
        pltpu.VMEM((NCAP,), jnp.int32),
        pltpu.VMEM((NCAP,), jnp.int32),
        pltpu.VMEM((4 * CHUNK, D), jnp.float32),
        pltpu.VMEM((BUCK, D), jnp.float32),
        pltpu.VMEM((BUCK, D), jnp.float32),
        pltpu.VMEM((BUCK, D), jnp.float32),
        pltpu.VMEM((BUCK, D), jnp.float32),
        pltpu.VMEM((16,), jnp.float32),
        pltpu.SemaphoreType.DMA,
        pltpu.SemaphoreType.DMA,
    ],
    compiler_params=_scparams,
)
def _sc_hop(y_hbm, bsrc_hbm, bslot_hbm, dis2_hbm, s_hbm, tk_hbm,
            yout_hbm, sout_hbm,
            src_v, slot_v, rows_v, acc, yown, d2own, sown, tk_v,
            gsem0, gsem1):
    c = lax.axis_index("c")
    s = lax.axis_index("s")
    w = s * 2 + c
    nsl = pl.ds(w * BUCK, BUCK)
    gsems = (gsem0, gsem1)

    pltpu.sync_copy(bsrc_hbm.at[w], src_v)
    pltpu.sync_copy(bslot_hbm.at[w], slot_v)
    pltpu.sync_copy(y_hbm.at[nsl], yown)
    pltpu.sync_copy(dis2_hbm.at[nsl], d2own)
    pltpu.sync_copy(s_hbm.at[nsl], sown)
    pltpu.sync_copy(tk_hbm, tk_v)

    def zero(i, carry):
        z16 = jnp.zeros((16,), jnp.float32)
        acc[i, pl.ds(0, 16)] = z16
        acc[i, pl.ds(16, 16)] = z16
        acc[i, pl.ds(32, 16)] = z16
        return carry

    lax.fori_loop(0, BUCK, zero, 0)

    def buf(eo, b):
        return rows_v.at[pl.ds((eo * 2 + b) * CHUNK, CHUNK)]

    def fire_g(i0, eo):
        for b in range(2):
            idx = src_v.at[pl.ds((i0 + b) * CHUNK, CHUNK)]
            pltpu.async_copy(y_hbm.at[idx], buf(eo, b), gsems[eo])

    def drain_g(eo):
        for b in range(2):
            pltpu.make_async_copy(y_hbm.at[pl.ds(0, CHUNK)], buf(eo, b),
                                  gsems[eo]).wait()

    def process(i0, eo):
        for b in range(2):
            base = i0 + b
            rbase = (eo * 2 + b) * CHUNK

            def grp16(j, carry):
                sv16 = slot_v[pl.ds(base * CHUNK + 16 * j, 16)]
                r0 = rbase + 16 * j
                for l in range(16):
                    slot = sv16[l]
                    r = r0 + l
                    plsc.addupdate(acc.at[slot, pl.ds(0, 16)],
                                   rows_v[r, pl.ds(0, 16)])
                    plsc.addupdate(acc.at[slot, pl.ds(16, 16)],
                                   rows_v[r, pl.ds(16, 16)])
                    plsc.addupdate(acc.at[slot, pl.ds(32, 16)],
                                   rows_v[r, pl.ds(32, 16)])
                return carry

            lax.fori_loop(0, CHUNK // 16, grp16, 0)

    fire_g(0, 0)

    def body(gp, carry):
        i0 = 4 * gp
        fire_g(i0 + 2, 1)
        drain_g(0)
        process(i0, 0)

        @pl.when(gp < NCAP_CH // 4 - 1)
        def _():
            fire_g(i0 + 4, 0)

        drain_g(1)
        process(i0 + 2, 1)
        return carry

    lax.fori_loop(0, NCAP_CH // 4, body, 0)

    tk = tk_v[pl.ds(0, 16)]

    def combine(i, carry):
        for t in range(3):
            lsl = pl.ds(16 * t, 16)
            z = acc[i, lsl] + yown[i, lsl]
            sown[i, lsl] = sown[i, lsl] + tk * z
            yown[i, lsl] = d2own[i, lsl] * z
        return carry

    lax.fori_loop(0, BUCK, combine, 0)
    pltpu.sync_copy(yown, yout_hbm.at[nsl])
    pltpu.sync_copy(sown, sout_hbm.at[nsl])


_BLK = 2048
_GRID = P // _BLK


def _mlp_body(x_ref, w1_ref, b1_ref, w2_ref, b2_ref, o_ref):
    x = x_ref[...]
    h1 = lax.dot_general(x, w1_ref[...], (((1,), (1,)), ((), ())),
                         preferred_element_type=jnp.float32)
    h1 = jax.nn.relu(h1 + b1_ref[...][None, :])
    h2 = lax.dot_general(h1, w2_ref[...], (((1,), (1,)), ((), ())),
                         preferred_element_type=jnp.float32)
    h2 = h2 + b2_ref[...][None, :]
    row = pl.program_id(0) * _BLK + lax.broadcasted_iota(jnp.int32, h2.shape, 0)
    o_ref[...] = jnp.where(row < N, h2, 0.0)


def _mlp(xp, w1, b1, w2p, b2p):
    return pl.pallas_call(
        _mlp_body,
        grid=(_GRID,),
        in_specs=[
            pl.BlockSpec((_BLK, 128), lambda i: (i, 0)),
            pl.BlockSpec((128, 128), lambda i: (0, 0)),
            pl.BlockSpec((128,), lambda i: (0,)),
            pl.BlockSpec((D, 128), lambda i: (0, 0)),
            pl.BlockSpec((D,), lambda i: (0,)),
        ],
        out_specs=pl.BlockSpec((_BLK, D), lambda i: (i, 0)),
        out_shape=jax.ShapeDtypeStruct((P, D), jnp.float32),
    )(xp, w1, b1, w2p, b2p)


def _pre_body(dg_ref, h_ref, o_y, o_dis, o_dis2):
    deg = dg_ref[:, 0:1]
    dis = lax.rsqrt(deg)
    dis2 = 1.0 / deg
    o_y[...] = dis * h_ref[...]
    o_dis[...] = jnp.broadcast_to(dis, o_dis.shape)
    o_dis2[...] = jnp.broadcast_to(dis2, o_dis2.shape)


def _pre(dg, h):
    spec = pl.BlockSpec((_BLK, D), lambda i: (i, 0))
    return pl.pallas_call(
        _pre_body,
        grid=(_GRID,),
        in_specs=[spec, spec],
        out_specs=[spec, spec, spec],
        out_shape=[jax.ShapeDtypeStruct((P, D), jnp.float32)] * 3,
    )(dg, h)


def _final_body(t0_ref, h_ref, dis_ref, s_ref, o_ref):
    o_ref[...] = t0_ref[0] * h_ref[...] + dis_ref[...] * s_ref[...]


def _final(t0, h, dis, s):
    spec = pl.BlockSpec((_BLK, D), lambda i: (i, 0))
    return pl.pallas_call(
        _final_body,
        grid=(_GRID,),
        in_specs=[pl.BlockSpec(memory_space=pltpu.SMEM), spec, spec, spec],
        out_specs=spec,
        out_shape=jax.ShapeDtypeStruct((P, D), jnp.float32),
    )(t0, h, dis, s)


def kernel(x, lin1_w, lin1_b, lin2_w, lin2_b, temp, edge_index):
    xp = jnp.zeros((P, 128), jnp.float32).at[:N].set(x)
    w2p = jnp.zeros((D, 128), jnp.float32).at[:DR].set(lin2_w)
    b2p = jnp.zeros((D,), jnp.float32).at[:DR].set(lin2_b)
    h = _mlp(xp, lin1_w, lin1_b, w2p, b2p)

    def _chunked(v):
        return (jnp.full((EPAD,), PAD_NODE, jnp.int32).at[:E].set(v)
                .reshape(ECHUNKS, CHUNK))

    srcc = _chunked(edge_index[0])
    dstc = _chunked(edge_index[1])
    bsrc, bslot = _sc_prep(srcc, dstc)

    zeros = jnp.zeros((P, D), jnp.float32)
    ones_all = jnp.ones((P, D), jnp.float32)
    ones_tbl = jnp.zeros((P, D), jnp.float32).at[:N].set(1.0)
    tks = jnp.broadcast_to(temp[:, None], (K + 1, 16))

    dg, _ = _sc_hop(ones_tbl, bsrc, bslot, ones_all, zeros, tks[0] * 0)
    y, dis, dis2 = _pre(dg, h)

    s = zeros
    for k in range(K):
        y, s = _sc_hop(y, bsrc, bslot, dis2, s, tks[k + 1])

    out = _final(temp[0:1], h, dis, s)
    return out[:N, :DR]


# speedup vs baseline: 2.9474x; 2.9474x over previous
import functools

import jax
import jax.numpy as jnp
from jax import lax
from jax.experimental import pallas as pl
from jax.experimental.pallas import tpu as pltpu
from jax.experimental.pallas import tpu_sc as plsc

N = 10000
E = 320000
D = 40
K = 10
P = 10240
NW = 32
NODES_PER_SC_TILE = P // 16
EPT = E // NW
CHUNK = 128
NBUF = 4
NCHUNK = 80
EPT_PAD = NCHUNK * CHUNK
NHALF = NCHUNK // (2 * NBUF)
PAD_NODE = N + 100

_mesh = plsc.VectorSubcoreMesh(core_axis_name="c", subcore_axis_name="s")


@functools.partial(
    pl.kernel,
    out_type=[jax.ShapeDtypeStruct((P, D), jnp.float32),
              jax.ShapeDtypeStruct((P, D), jnp.float32)],
    mesh=_mesh,
    scratch_types=[
        pltpu.VMEM((NCHUNK, CHUNK), jnp.int32),
        pltpu.VMEM((NCHUNK, CHUNK), jnp.int32),
        pltpu.VMEM((2 * NBUF * CHUNK, D), jnp.float32),
        pltpu.VMEM_SHARED((P, D), jnp.float32),
        pltpu.SemaphoreType.DMA,
        pltpu.SemaphoreType.DMA,
        pltpu.SemaphoreType.DMA,
        pltpu.SemaphoreType.DMA,
    ],
    compiler_params=pltpu.CompilerParams(use_tc_tiling_on_sc=False),
)
def _sc_scatter(y_hbm, src_hbm, dst_hbm, zeros_hbm, out0_hbm, out1_hbm,
                src_v, dst_v, rows_v, acc_sh, gsem0, gsem1, ssem0, ssem1):
    c = lax.axis_index("c")
    s = lax.axis_index("s")
    wid = s * 2 + c
    nslice = pl.ds(s * NODES_PER_SC_TILE, NODES_PER_SC_TILE)
    pltpu.sync_copy(zeros_hbm.at[nslice], acc_sh.at[nslice])
    pltpu.sync_copy(src_hbm.at[wid], src_v)
    pltpu.sync_copy(dst_hbm.at[wid], dst_v)
    plsc.subcore_barrier()

    rows0 = rows_v.at[pl.ds(0, CHUNK)]

    def body(j, carry):
        pltpu.async_copy(y_hbm.at[src_v.at[j]], rows0, gsem0).wait()
        pltpu.sync_copy(rows0, acc_sh.at[dst_v.at[j]], add=True)
        return carry

    lax.fori_loop(0, NCHUNK, body, 0)
    plsc.subcore_barrier()

    @pl.when(c == 0)
    def _():
        pltpu.sync_copy(acc_sh.at[nslice], out0_hbm.at[nslice])

    @pl.when(c == 1)
    def _():
        pltpu.sync_copy(acc_sh.at[nslice], out1_hbm.at[nslice])


_BLK = 2048
_GRID = P // _BLK


def _mlp_body(x_ref, w1_ref, b1_ref, w2_ref, b2_ref, o_ref):
    x = x_ref[...]
    h1 = lax.dot_general(x, w1_ref[...], (((1,), (1,)), ((), ())),
                         preferred_element_type=jnp.float32)
    h1 = jax.nn.relu(h1 + b1_ref[...][None, :])
    h2 = lax.dot_general(h1, w2_ref[...], (((1,), (1,)), ((), ())),
                         preferred_element_type=jnp.float32)
    h2 = h2 + b2_ref[...][None, :]
    row = pl.program_id(0) * _BLK + lax.broadcasted_iota(jnp.int32, h2.shape, 0)
    o_ref[...] = jnp.where(row < N, h2, 0.0)


def _mlp(xp, w1, b1, w2, b2):
    return pl.pallas_call(
        _mlp_body,
        grid=(_GRID,),
        in_specs=[
            pl.BlockSpec((_BLK, 128), lambda i: (i, 0)),
            pl.BlockSpec((128, 128), lambda i: (0, 0)),
            pl.BlockSpec((128,), lambda i: (0,)),
            pl.BlockSpec((D, 128), lambda i: (0, 0)),
            pl.BlockSpec((D,), lambda i: (0,)),
        ],
        out_specs=pl.BlockSpec((_BLK, D), lambda i: (i, 0)),
        out_shape=jax.ShapeDtypeStruct((P, D), jnp.float32),
    )(xp, w1, b1, w2, b2)


def _pre_body(pd0_ref, pd1_ref, h_ref, o_y, o_dis, o_dis2):
    deg = pd0_ref[:, 0:1] + pd1_ref[:, 0:1] + 1.0
    dis = lax.rsqrt(deg)
    dis2 = 1.0 / deg
    o_y[...] = dis * h_ref[...]
    o_dis[...] = jnp.broadcast_to(dis, o_dis.shape)
    o_dis2[...] = jnp.broadcast_to(dis2, o_dis2.shape)


def _pre(pd0, pd1, h):
    spec = pl.BlockSpec((_BLK, D), lambda i: (i, 0))
    return pl.pallas_call(
        _pre_body,
        grid=(_GRID,),
        in_specs=[spec, spec, spec],
        out_specs=[spec, spec, spec],
        out_shape=[jax.ShapeDtypeStruct((P, D), jnp.float32)] * 3,
    )(pd0, pd1, h)


def _combine_body(tj_ref, p0_ref, p1_ref, y_ref, s_ref, dis2_ref, o_y, o_s):
    z = p0_ref[...] + p1_ref[...] + y_ref[...]
    o_s[...] = s_ref[...] + tj_ref[0] * z
    o_y[...] = dis2_ref[...] * z


def _combine(tj, p0, p1, y, s, dis2):
    spec = pl.BlockSpec((_BLK, D), lambda i: (i, 0))
    return pl.pallas_call(
        _combine_body,
        grid=(_GRID,),
        in_specs=[pl.BlockSpec(memory_space=pltpu.SMEM),
                  spec, spec, spec, spec, spec],
        out_specs=[spec, spec],
        out_shape=[jax.ShapeDtypeStruct((P, D), jnp.float32)] * 2,
    )(tj, p0, p1, y, s, dis2)


def _final_body(t0_ref, h_ref, dis_ref, s_ref, o_ref):
    o_ref[...] = t0_ref[0] * h_ref[...] + dis_ref[...] * s_ref[...]


def _final(t0, h, dis, s):
    spec = pl.BlockSpec((_BLK, D), lambda i: (i, 0))
    return pl.pallas_call(
        _final_body,
        grid=(_GRID,),
        in_specs=[pl.BlockSpec(memory_space=pltpu.SMEM), spec, spec, spec],
        out_specs=spec,
        out_shape=jax.ShapeDtypeStruct((P, D), jnp.float32),
    )(t0, h, dis, s)


def kernel(x, lin1_w, lin1_b, lin2_w, lin2_b, temp, edge_index):
    xp = jnp.zeros((P, 128), jnp.float32).at[:N].set(x)
    h = _mlp(xp, lin1_w, lin1_b, lin2_w, lin2_b)

    def _tile_idx(v):
        v2 = v.reshape(NW, EPT)
        vp = jnp.full((NW, EPT_PAD), PAD_NODE, jnp.int32).at[:, :EPT].set(v2)
        return vp.reshape(NW, NCHUNK, CHUNK)

    src_p = _tile_idx(edge_index[0])
    dst_p = _tile_idx(edge_index[1])

    zeros = jnp.zeros((P, D), jnp.float32)
    ones_tbl = jnp.zeros((P, D), jnp.float32).at[:N].set(1.0)

    pd0, pd1 = _sc_scatter(ones_tbl, src_p, dst_p, zeros)
    y, dis, dis2 = _pre(pd0, pd1, h)

    s = zeros
    for k in range(K):
        p0, p1 = _sc_scatter(y, src_p, dst_p, zeros)
        y, s = _combine(temp[k + 1:k + 2], p0, p1, y, s, dis2)

    out = _final(temp[0:1], h, dis, s)
    return out[:N]


# speedup vs baseline: 3.8718x; 1.3136x over previous
import functools

import jax
import jax.numpy as jnp
from jax import lax
from jax.experimental import pallas as pl
from jax.experimental.pallas import tpu as pltpu
from jax.experimental.pallas import tpu_sc as plsc

N = 10000
E = 320000
D = 40
K = 10
P = 10240
NW = 32
NODES_PER_SC_TILE = P // 16
EPT = E // NW
CHUNK = 128
NCHUNK = 79
EPT_PAD = NCHUNK * CHUNK
PAD_NODE = N + 100

_mesh = plsc.VectorSubcoreMesh(core_axis_name="c", subcore_axis_name="s")


@functools.partial(
    pl.kernel,
    out_type=[jax.ShapeDtypeStruct((P, D), jnp.float32),
              jax.ShapeDtypeStruct((P, D), jnp.float32)],
    mesh=_mesh,
    scratch_types=[
        pltpu.VMEM((NCHUNK, CHUNK), jnp.int32),
        pltpu.VMEM((NCHUNK, CHUNK), jnp.int32),
        pltpu.VMEM((CHUNK, D), jnp.float32),
        pltpu.VMEM_SHARED((P, D), jnp.float32),
        pltpu.SemaphoreType.DMA,
    ],
    compiler_params=pltpu.CompilerParams(use_tc_tiling_on_sc=False),
)
def _sc_scatter(y_hbm, src_hbm, dst_hbm, zeros_hbm, out0_hbm, out1_hbm,
                src_v, dst_v, rows_v, acc_sh, sem):
    c = lax.axis_index("c")
    s = lax.axis_index("s")
    wid = s * 2 + c
    nslice = pl.ds(s * NODES_PER_SC_TILE, NODES_PER_SC_TILE)
    pltpu.sync_copy(zeros_hbm.at[nslice], acc_sh.at[nslice])
    pltpu.sync_copy(src_hbm.at[wid], src_v)
    pltpu.sync_copy(dst_hbm.at[wid], dst_v)
    plsc.subcore_barrier()

    def body(j, carry):
        pltpu.async_copy(y_hbm.at[src_v.at[j]], rows_v, sem).wait()
        pltpu.sync_copy(rows_v, acc_sh.at[dst_v.at[j]], add=True)
        return carry

    lax.fori_loop(0, NCHUNK, body, 0)
    plsc.subcore_barrier()

    @pl.when(c == 0)
    def _():
        pltpu.sync_copy(acc_sh.at[nslice], out0_hbm.at[nslice])

    @pl.when(c == 1)
    def _():
        pltpu.sync_copy(acc_sh.at[nslice], out1_hbm.at[nslice])


_BLK = 2048
_GRID = P // _BLK


def _mlp_body(x_ref, w1_ref, b1_ref, w2_ref, b2_ref, o_ref):
    x = x_ref[...]
    h1 = lax.dot_general(x, w1_ref[...], (((1,), (1,)), ((), ())),
                         preferred_element_type=jnp.float32)
    h1 = jax.nn.relu(h1 + b1_ref[...][None, :])
    h2 = lax.dot_general(h1, w2_ref[...], (((1,), (1,)), ((), ())),
                         preferred_element_type=jnp.float32)
    h2 = h2 + b2_ref[...][None, :]
    row = pl.program_id(0) * _BLK + lax.broadcasted_iota(jnp.int32, h2.shape, 0)
    o_ref[...] = jnp.where(row < N, h2, 0.0)


def _mlp(xp, w1, b1, w2, b2):
    return pl.pallas_call(
        _mlp_body,
        grid=(_GRID,),
        in_specs=[
            pl.BlockSpec((_BLK, 128), lambda i: (i, 0)),
            pl.BlockSpec((128, 128), lambda i: (0, 0)),
            pl.BlockSpec((128,), lambda i: (0,)),
            pl.BlockSpec((D, 128), lambda i: (0, 0)),
            pl.BlockSpec((D,), lambda i: (0,)),
        ],
        out_specs=pl.BlockSpec((_BLK, D), lambda i: (i, 0)),
        out_shape=jax.ShapeDtypeStruct((P, D), jnp.float32),
    )(xp, w1, b1, w2, b2)


def _pre_body(pd0_ref, pd1_ref, h_ref, o_y, o_dis, o_dis2):
    deg = pd0_ref[:, 0:1] + pd1_ref[:, 0:1] + 1.0
    dis = lax.rsqrt(deg)
    dis2 = 1.0 / deg
    o_y[...] = dis * h_ref[...]
    o_dis[...] = jnp.broadcast_to(dis, o_dis.shape)
    o_dis2[...] = jnp.broadcast_to(dis2, o_dis2.shape)


def _pre(pd0, pd1, h):
    spec = pl.BlockSpec((_BLK, D), lambda i: (i, 0))
    return pl.pallas_call(
        _pre_body,
        grid=(_GRID,),
        in_specs=[spec, spec, spec],
        out_specs=[spec, spec, spec],
        out_shape=[jax.ShapeDtypeStruct((P, D), jnp.float32)] * 3,
    )(pd0, pd1, h)


def _combine_body(tj_ref, p0_ref, p1_ref, y_ref, s_ref, dis2_ref, o_y, o_s):
    z = p0_ref[...] + p1_ref[...] + y_ref[...]
    o_s[...] = s_ref[...] + tj_ref[0] * z
    o_y[...] = dis2_ref[...] * z


def _combine(tj, p0, p1, y, s, dis2):
    spec = pl.BlockSpec((_BLK, D), lambda i: (i, 0))
    return pl.pallas_call(
        _combine_body,
        grid=(_GRID,),
        in_specs=[pl.BlockSpec(memory_space=pltpu.SMEM),
                  spec, spec, spec, spec, spec],
        out_specs=[spec, spec],
        out_shape=[jax.ShapeDtypeStruct((P, D), jnp.float32)] * 2,
    )(tj, p0, p1, y, s, dis2)


def _final_body(t0_ref, h_ref, dis_ref, s_ref, o_ref):
    o_ref[...] = t0_ref[0] * h_ref[...] + dis_ref[...] * s_ref[...]


def _final(t0, h, dis, s):
    spec = pl.BlockSpec((_BLK, D), lambda i: (i, 0))
    return pl.pallas_call(
        _final_body,
        grid=(_GRID,),
        in_specs=[pl.BlockSpec(memory_space=pltpu.SMEM), spec, spec, spec],
        out_specs=spec,
        out_shape=jax.ShapeDtypeStruct((P, D), jnp.float32),
    )(t0, h, dis, s)


def kernel(x, lin1_w, lin1_b, lin2_w, lin2_b, temp, edge_index):
    xp = jnp.zeros((P, 128), jnp.float32).at[:N].set(x)
    h = _mlp(xp, lin1_w, lin1_b, lin2_w, lin2_b)

    def _tile_idx(v):
        v2 = v.reshape(NW, EPT)
        vp = jnp.full((NW, EPT_PAD), PAD_NODE, jnp.int32).at[:, :EPT].set(v2)
        return vp.reshape(NW, NCHUNK, CHUNK)

    src_p = _tile_idx(edge_index[0])
    dst_p = _tile_idx(edge_index[1])

    zeros = jnp.zeros((P, D), jnp.float32)
    ones_tbl = jnp.zeros((P, D), jnp.float32).at[:N].set(1.0)

    pd0, pd1 = _sc_scatter(ones_tbl, src_p, dst_p, zeros)
    y, dis, dis2 = _pre(pd0, pd1, h)

    s = zeros
    for k in range(K):
        p0, p1 = _sc_scatter(y, src_p, dst_p, zeros)
        y, s = _combine(temp[k + 1:k + 2], p0, p1, y, s, dis2)

    out = _final(temp[0:1], h, dis, s)
    return out[:N]
